# baseline (speedup 1.0000x reference)
import functools

import jax
import jax.numpy as jnp
from jax import lax
from jax.experimental import pallas as pl
from jax.experimental.pallas import tpu as pltpu
from jax.experimental.pallas import tpu_sc as plsc

_NC = 2
_NS = 16
_NW = _NC * _NS
_NBUF = 8


@functools.lru_cache(maxsize=None)
def _make_gather(N, S, D):
    per_w = N // _NW
    nblk = per_w // _NBUF
    mesh = plsc.VectorSubcoreMesh(core_axis_name="c", subcore_axis_name="s")

    scratch = [pltpu.VMEM((per_w, S), jnp.int32)]
    scratch += [pltpu.VMEM((S, D), jnp.float32) for _ in range(_NBUF)]
    scratch += [pltpu.SemaphoreType.DMA for _ in range(_NBUF)]

    @functools.partial(
        pl.kernel,
        mesh=mesh,
        out_type=jax.ShapeDtypeStruct((N, S, D), jnp.float32),
        scratch_types=scratch,
        compiler_params=pltpu.CompilerParams(use_tc_tiling_on_sc=True),
    )
    def k(idx_hbm, table_hbm, out_hbm, idx_v, *rest):
        bufs = rest[:_NBUF]
        sems = rest[_NBUF:]
        wid = lax.axis_index("s") * _NC + lax.axis_index("c")
        b0 = wid * per_w
        pltpu.sync_copy(idx_hbm.at[wid], idx_v)

        def start_gather(g, b):
            pltpu.make_async_copy(table_hbm.at[idx_v.at[g]], bufs[b], sems[b]).start()

        def wait_gather(g, b):
            pltpu.make_async_copy(table_hbm.at[idx_v.at[g]], bufs[b], sems[b]).wait()

        for b in range(_NBUF):
            start_gather(b, b)

        def body(blk, carry):
            for b in range(_NBUF):
                g = blk * _NBUF + b
                wait_gather(g, b)
                pltpu.sync_copy(bufs[b], out_hbm.at[b0 + g])
                start_gather(g + _NBUF, b)
            return carry

        lax.fori_loop(0, nblk - 1, body, 0)

        for b in range(_NBUF):
            g = (nblk - 1) * _NBUF + b
            wait_gather(g, b)
            pltpu.sync_copy(bufs[b], out_hbm.at[b0 + g])

    return k


def kernel(x, table):
    N, S = x.shape
    D = table.shape[1]
    idx = x.reshape(_NW, N // _NW, S).astype(jnp.int32)
    return _make_gather(N, S, D)(idx, table)


# speedup vs baseline: 1.7794x; 1.7794x over previous
import functools

import jax
import jax.numpy as jnp
from jax import lax
from jax.experimental import pallas as pl
from jax.experimental.pallas import tpu as pltpu
from jax.experimental.pallas import tpu_sc as plsc

_NC = 2
_NS = 16
_NW = _NC * _NS
_J = 2
_NBUF = 3


@functools.lru_cache(maxsize=None)
def _make_gather(N, S, D):
    per_w = N // _NW
    nchunk = S // _J
    mesh = plsc.VectorSubcoreMesh(core_axis_name="c", subcore_axis_name="s")

    scratch = [pltpu.VMEM((nchunk, _J, per_w), jnp.int32)]
    scratch += [pltpu.VMEM((_J, per_w, D), jnp.float32) for _ in range(_NBUF)]
    scratch += [pltpu.SemaphoreType.DMA for _ in range(_NBUF)]

    @functools.partial(
        pl.kernel,
        mesh=mesh,
        out_type=jax.ShapeDtypeStruct((S, N, D), jnp.float32),
        scratch_types=scratch,
        compiler_params=pltpu.CompilerParams(use_tc_tiling_on_sc=True),
    )
    def k(idx_hbm, table_hbm, out_hbm, idx_v, *rest):
        bufs = rest[:_NBUF]
        sems = rest[_NBUF:]
        wid = lax.axis_index("s") * _NC + lax.axis_index("c")
        b0 = wid * per_w
        pltpu.sync_copy(idx_hbm.at[wid], idx_v)

        def gather(g, jj, b):
            return pltpu.make_async_copy(
                table_hbm.at[idx_v.at[g, jj]], bufs[b].at[jj], sems[b])

        def start_gather(g, b):
            for jj in range(_J):
                gather(g, jj, b).start()

        def wait_gather(g, b):
            for jj in range(_J):
                gather(g, jj, b).wait()

        def put_out(g, b):
            pltpu.sync_copy(
                bufs[b], out_hbm.at[pl.ds(g * _J, _J), pl.ds(b0, per_w)])

        for b in range(_NBUF):
            start_gather(b, b)

        nblk = (nchunk - _NBUF - 1) // _NBUF

        def body(blk, carry):
            for b in range(_NBUF):
                g = blk * _NBUF + b
                wait_gather(g, b)
                put_out(g, b)
                start_gather(g + _NBUF, b)
            return carry

        lax.fori_loop(0, nblk, body, 0)

        for g in range(nblk * _NBUF, nchunk):
            b = g % _NBUF
            wait_gather(g, b)
            put_out(g, b)
            if g + _NBUF < nchunk:
                start_gather(g + _NBUF, b)

    return k


def kernel(x, table):
    N, S = x.shape
    D = table.shape[1]
    per_w = N // _NW
    idx = (x.reshape(_NW, per_w, S // _J, _J)
             .transpose(0, 2, 3, 1).astype(jnp.int32))
    out = _make_gather(N, S, D)(idx, table)
    return out.transpose(1, 0, 2)


# speedup vs baseline: 1.7937x; 1.0080x over previous
import functools

import jax
import jax.numpy as jnp
from jax import lax
from jax.experimental import pallas as pl
from jax.experimental.pallas import tpu as pltpu
from jax.experimental.pallas import tpu_sc as plsc

_NC = 2
_NS = 16
_NW = _NC * _NS
_NBUF = 5


@functools.lru_cache(maxsize=None)
def _make_gather(N, S, D):
    per_w = N // _NW
    nblk = S // _NBUF
    mesh = plsc.VectorSubcoreMesh(core_axis_name="c", subcore_axis_name="s")

    scratch = [pltpu.VMEM((S, per_w), jnp.int32)]
    scratch += [pltpu.VMEM((per_w, D), jnp.float32) for _ in range(_NBUF)]
    scratch += [pltpu.SemaphoreType.DMA for _ in range(_NBUF)]

    @functools.partial(
        pl.kernel,
        mesh=mesh,
        out_type=jax.ShapeDtypeStruct((S, N, D), jnp.float32),
        scratch_types=scratch,
        compiler_params=pltpu.CompilerParams(use_tc_tiling_on_sc=True),
    )
    def k(idx_hbm, table_hbm, out_hbm, idx_v, *rest):
        bufs = rest[:_NBUF]
        sems = rest[_NBUF:]
        wid = lax.axis_index("s") * _NC + lax.axis_index("c")
        b0 = wid * per_w
        pltpu.sync_copy(idx_hbm.at[wid], idx_v)

        def start_gather(j, b):
            pltpu.make_async_copy(table_hbm.at[idx_v.at[j]], bufs[b], sems[b]).start()

        def wait_gather(j, b):
            pltpu.make_async_copy(table_hbm.at[idx_v.at[j]], bufs[b], sems[b]).wait()

        def put_out(j, b):
            pltpu.sync_copy(bufs[b], out_hbm.at[j, pl.ds(b0, per_w)])

        for b in range(_NBUF):
            start_gather(b, b)

        def body(blk, carry):
            for b in range(_NBUF):
                j = blk * _NBUF + b
                wait_gather(j, b)
                put_out(j, b)
                start_gather(j + _NBUF, b)
            return carry

        lax.fori_loop(0, nblk - 1, body, 0)

        for b in range(_NBUF):
            j = (nblk - 1) * _NBUF + b
            wait_gather(j, b)
            put_out(j, b)

    return k


def kernel(x, table):
    N, S = x.shape
    D = table.shape[1]
    per_w = N // _NW
    idx = x.reshape(_NW, per_w, S).transpose(0, 2, 1).astype(jnp.int32)
    out = _make_gather(N, S, D)(idx, table)
    return out.transpose(1, 0, 2)


# speedup vs baseline: 1.8074x; 1.0077x over previous
import functools

import jax
import jax.numpy as jnp
from jax import lax
from jax.experimental import pallas as pl
from jax.experimental.pallas import tpu as pltpu
from jax.experimental.pallas import tpu_sc as plsc

_NC = 2
_NS = 16
_NW = _NC * _NS
_NBUF = 5


@functools.lru_cache(maxsize=None)
def _make_gather(N, S, D):
    per_w = N // _NW
    nblk = S // _NBUF
    mesh = plsc.VectorSubcoreMesh(core_axis_name="c", subcore_axis_name="s")

    scratch = [pltpu.VMEM((S, per_w), jnp.int32)]
    scratch += [pltpu.VMEM((per_w, D), jnp.float32) for _ in range(_NBUF)]
    scratch += [pltpu.SemaphoreType.DMA for _ in range(_NBUF)]

    @functools.partial(
        pl.kernel,
        mesh=mesh,
        out_type=jax.ShapeDtypeStruct((S, N, D), jnp.float32),
        scratch_types=scratch,
    )
    def k(idx_hbm, table_hbm, out_hbm, idx_v, *rest):
        bufs = rest[:_NBUF]
        sems = rest[_NBUF:]
        wid = lax.axis_index("s") * _NC + lax.axis_index("c")
        b0 = wid * per_w
        pltpu.sync_copy(idx_hbm.at[:, pl.ds(b0, per_w)], idx_v)

        def start_gather(j, b):
            pltpu.make_async_copy(table_hbm.at[idx_v.at[j]], bufs[b], sems[b]).start()

        def wait_gather(j, b):
            pltpu.make_async_copy(table_hbm.at[idx_v.at[j]], bufs[b], sems[b]).wait()

        def put_out(j, b):
            pltpu.sync_copy(bufs[b], out_hbm.at[j, pl.ds(b0, per_w)])

        for b in range(_NBUF):
            start_gather(b, b)

        def body(blk, carry):
            for b in range(_NBUF):
                j = blk * _NBUF + b
                wait_gather(j, b)
                put_out(j, b)
                start_gather(j + _NBUF, b)
            return carry

        lax.fori_loop(0, nblk - 1, body, 0)

        for b in range(_NBUF):
            j = (nblk - 1) * _NBUF + b
            wait_gather(j, b)
            put_out(j, b)

    return k


def kernel(x, table):
    N, S = x.shape
    D = table.shape[1]
    idx = x.T.astype(jnp.int32)
    out = _make_gather(N, S, D)(idx, table)
    return out.transpose(1, 0, 2)
